# baseline (speedup 1.0000x reference)
import functools

import jax
import jax.numpy as jnp
from jax import lax
from jax.experimental import pallas as pl
from jax.experimental.pallas import tpu as pltpu
from jax.experimental.pallas import tpu_sc as plsc

PLAYER_DIM = 64
VENUE_DIM = 32
TEAM_DIM = 32
B = 16384

NC = 2
NS = 16
NW = NC * NS
BPW = B // NW
CHUNK = 128
NCHUNK = BPW // CHUNK


def _body(player_t, venue_t, team_t,
          batter_i, bowler_i, non_striker_i, venue_i, batting_i, bowling_i,
          batter_o, bowler_o, non_striker_o, venue_o, batting_o, bowling_o,
          idx_v, rows_p0, rows_p1, rows_s0, rows_s1, sem_g, sem_w):
    wid = lax.axis_index("s") * NC + lax.axis_index("c")
    row0 = wid * BPW

    lookups = [
        (player_t, batter_i, batter_o, rows_p0),
        (player_t, bowler_i, bowler_o, rows_p1),
        (player_t, non_striker_i, non_striker_o, rows_p0),
        (venue_t, venue_i, venue_o, rows_s0),
        (team_t, batting_i, batting_o, rows_s1),
        (team_t, bowling_i, bowling_o, rows_s0),
    ]

    for k, (_t, idx, _o, _r) in enumerate(lookups):
        pltpu.sync_copy(idx.at[pl.ds(wid * NCHUNK, NCHUNK)], idx_v.at[k])

    def fire_gather(k):
        table, _idx, _out, rows = lookups[k]
        cps = []
        for c in range(NCHUNK):
            cps.append(pltpu.async_copy(
                table.at[idx_v.at[k, c]],
                rows.at[pl.ds(c * CHUNK, CHUNK)], sem_g))
        return cps

    gathers = [fire_gather(0)]
    writes = [None] * 6
    for k in range(6):
        for cp in gathers[k]:
            cp.wait()
        if k >= 1:
            writes[k - 1].wait()
        if k + 1 < 6:
            gathers.append(fire_gather(k + 1))
        _table, _idx, out, rows = lookups[k]
        writes[k] = pltpu.async_copy(rows, out.at[pl.ds(row0, BPW)], sem_w)
    writes[5].wait()


@jax.jit
def _run(player_t, venue_t, team_t, batter_i, bowler_i, non_striker_i,
         venue_i, batting_i, bowling_i):
    f32 = jnp.float32
    out_type = (
        jax.ShapeDtypeStruct((B, PLAYER_DIM), f32),
        jax.ShapeDtypeStruct((B, PLAYER_DIM), f32),
        jax.ShapeDtypeStruct((B, PLAYER_DIM), f32),
        jax.ShapeDtypeStruct((B, VENUE_DIM), f32),
        jax.ShapeDtypeStruct((B, TEAM_DIM), f32),
        jax.ShapeDtypeStruct((B, TEAM_DIM), f32),
    )
    mesh = plsc.VectorSubcoreMesh(
        core_axis_name="c", subcore_axis_name="s",
        num_cores=NC, num_subcores=NS)
    kern = pl.kernel(
        _body,
        out_type,
        mesh=mesh,
        compiler_params=pltpu.CompilerParams(use_tc_tiling_on_sc=False),
        scratch_types=[
            pltpu.VMEM((6, NCHUNK, CHUNK), jnp.int32),
            pltpu.VMEM((BPW, PLAYER_DIM), f32),
            pltpu.VMEM((BPW, PLAYER_DIM), f32),
            pltpu.VMEM((BPW, VENUE_DIM), f32),
            pltpu.VMEM((BPW, TEAM_DIM), f32),
            pltpu.SemaphoreType.DMA,
            pltpu.SemaphoreType.DMA,
        ],
    )
    return kern(player_t, venue_t, team_t, batter_i, bowler_i,
                non_striker_i, venue_i, batting_i, bowling_i)


def kernel(player_table, venue_table, team_table, batter_idx, bowler_idx,
           non_striker_idx, venue_idx, batting_team_idx, bowling_team_idx):
    def prep(i):
        return i.astype(jnp.int32).reshape(B // CHUNK, CHUNK)
    return _run(player_table, venue_table, team_table,
                prep(batter_idx), prep(bowler_idx), prep(non_striker_idx),
                prep(venue_idx), prep(batting_team_idx), prep(bowling_team_idx))
